# baseline (speedup 1.0000x reference)
import jax
import jax.numpy as jnp
from jax.experimental import pallas as pl

_B, _M, _N, _C = 4, 2048, 4096, 128
_NC, _H1, _H2, _OUT = 20, 128, 256, 128
_TN = 2048

_HIGH = jax.lax.Precision.HIGHEST


def _dot(a, b):
    return jnp.dot(a.astype(jnp.bfloat16), b.astype(jnp.bfloat16),
                   preferred_element_type=jnp.float32)


def _semantic_body(f_ref, w1_ref, b1_ref, w2_ref, b2_ref, lg_ref, p_ref):
    f = f_ref[0]
    h = jnp.maximum(_dot(f, w1_ref[...]) + b1_ref[...], 0.0)
    lg = _dot(h, w2_ref[...]) + b2_ref[...]
    lg_ref[0] = lg
    e = jnp.exp(lg - jnp.max(lg, axis=-1, keepdims=True))
    p_ref[0] = e / jnp.sum(e, axis=-1, keepdims=True)


def _fuse_body(tgt_ref, src_ref, f_ref, p_ref, u1a_ref, u1b_ref, bu1_ref,
               u2_ref, bu2_ref, out_ref):
    t = tgt_ref[0]
    s = src_ref[0]
    t2 = jnp.sum(t * t, axis=1, keepdims=True)
    s2 = jnp.sum(s * s, axis=0, keepdims=True) * 0.25
    cross = _dot(t, s)
    sq = (t2 + s2) + cross

    inf = jnp.float32(jnp.inf)
    zero = jnp.float32(0.0)
    one = jnp.float32(1.0)

    def _weights(v0, v1, v2):
        d0 = jnp.sqrt(jnp.maximum(v0, 1e-12))
        d1 = jnp.sqrt(jnp.maximum(v1, 1e-12))
        d2 = jnp.sqrt(jnp.maximum(v2, 1e-12))
        nd0, nd1, nd2 = -d0, -d1, -d2
        mx = jnp.maximum(nd0, jnp.maximum(nd1, nd2))
        e0 = jnp.exp(nd0 - mx)
        e1 = jnp.exp(nd1 - mx)
        e2 = jnp.exp(nd2 - mx)
        z = e0 + e1 + e2
        return e0 / z, e1 / z, e2 / z

    vmin0 = jnp.min(sq, axis=1, keepdims=True)
    gt0 = sq > vmin0
    vmin1 = jnp.min(jnp.where(gt0, sq, inf), axis=1, keepdims=True)
    gt1 = sq > vmin1
    vmin2 = jnp.min(jnp.where(gt1, sq, inf), axis=1, keepdims=True)
    le2 = sq <= vmin2
    total = jnp.sum(le2.astype(jnp.float32))
    w0, w1, w2 = _weights(vmin0, vmin1, vmin2)
    wsum_fast = jnp.where(gt0, jnp.where(gt1, jnp.where(le2, w2, zero), w1),
                          w0)
    asum_fast = jnp.where(le2, one, zero)

    fused_fast = _dot(wsum_fast, f_ref[0])
    semw_fast = _dot(asum_fast, p_ref[0])

    def _exact_path(_):
        iota = jax.lax.broadcasted_iota(jnp.int32, sq.shape, 1)
        i0 = jnp.min(jnp.where(sq == vmin0, iota, _M), axis=1, keepdims=True)
        m0 = iota == i0
        v1 = jnp.min(jnp.where(m0, inf, sq), axis=1, keepdims=True)
        i1 = jnp.min(jnp.where((sq == v1) & ~m0, iota, _M),
                     axis=1, keepdims=True)
        m01 = m0 | (iota == i1)
        v2 = jnp.min(jnp.where(m01, inf, sq), axis=1, keepdims=True)
        i2 = jnp.min(jnp.where((sq == v2) & ~m01, iota, _M),
                     axis=1, keepdims=True)
        x0, x1, x2 = _weights(vmin0, v1, v2)
        ws = (jnp.where(iota == i0, x0, zero)
              + jnp.where(iota == i1, x1, zero)
              + jnp.where(iota == i2, x2, zero))
        asm = jnp.where(ws > zero, one, zero)
        return _dot(ws, f_ref[0]), _dot(asm, p_ref[0])

    fused, semw3 = jax.lax.cond(
        total == jnp.float32(3 * _TN), lambda _: (fused_fast, semw_fast),
        _exact_path, None)
    semw = semw3 * jnp.float32(1.0 / 3.0)

    h = jnp.maximum(
        _dot(fused, u1a_ref[...]) + _dot(semw, u1b_ref[...])
        + bu1_ref[...], 0.0)
    out_ref[0] = _dot(h, u2_ref[...]) + bu2_ref[...]


def kernel(src_points, tgt_points, src_features, W1, b1, W2, b2,
           U1, bu1, U2, bu2):
    B, M, _ = src_points.shape
    N = tgt_points.shape[1]
    C = src_features.shape[-1]
    NC = W2.shape[-1]
    H2 = U1.shape[-1]
    OUT = U2.shape[-1]

    logits, p = pl.pallas_call(
        _semantic_body,
        grid=(B,),
        in_specs=[
            pl.BlockSpec((1, M, C), lambda b: (b, 0, 0)),
            pl.BlockSpec((C, _H1), lambda b: (0, 0)),
            pl.BlockSpec((1, _H1), lambda b: (0, 0)),
            pl.BlockSpec((_H1, NC), lambda b: (0, 0)),
            pl.BlockSpec((1, NC), lambda b: (0, 0)),
        ],
        out_specs=[
            pl.BlockSpec((1, M, NC), lambda b: (b, 0, 0)),
            pl.BlockSpec((1, M, NC), lambda b: (b, 0, 0)),
        ],
        out_shape=[
            jax.ShapeDtypeStruct((B, M, NC), jnp.float32),
            jax.ShapeDtypeStruct((B, M, NC), jnp.float32),
        ],
    )(src_features, W1, b1.reshape(1, -1), W2, b2.reshape(1, -1))

    tgt_pad = jnp.concatenate(
        [tgt_points, jnp.zeros((B, N, 5), jnp.float32)], axis=2)
    src_pad = jnp.concatenate(
        [src_points.transpose(0, 2, 1) * -2.0,
         jnp.zeros((B, 5, M), jnp.float32)], axis=1)

    upsampled = pl.pallas_call(
        _fuse_body,
        grid=(B, N // _TN),
        in_specs=[
            pl.BlockSpec((1, _TN, 8), lambda b, j: (b, j, 0)),
            pl.BlockSpec((1, 8, M), lambda b, j: (b, 0, 0)),
            pl.BlockSpec((1, M, C), lambda b, j: (b, 0, 0)),
            pl.BlockSpec((1, M, NC), lambda b, j: (b, 0, 0)),
            pl.BlockSpec((C, H2), lambda b, j: (0, 0)),
            pl.BlockSpec((NC, H2), lambda b, j: (0, 0)),
            pl.BlockSpec((1, H2), lambda b, j: (0, 0)),
            pl.BlockSpec((H2, OUT), lambda b, j: (0, 0)),
            pl.BlockSpec((1, OUT), lambda b, j: (0, 0)),
        ],
        out_specs=pl.BlockSpec((1, _TN, OUT), lambda b, j: (b, j, 0)),
        out_shape=jax.ShapeDtypeStruct((B, N, OUT), jnp.float32),
    )(tgt_pad, src_pad, src_features, p, U1[:C], U1[C:],
      bu1.reshape(1, -1), U2, bu2.reshape(1, -1))

    return (upsampled, logits)


# speedup vs baseline: 1.0349x; 1.0349x over previous
import jax
import jax.numpy as jnp
from jax.experimental import pallas as pl

_B, _M, _N, _C = 4, 2048, 4096, 128
_NC, _H1, _H2, _OUT = 20, 128, 256, 128
_TN = 1024

_HIGH = jax.lax.Precision.HIGHEST


def _dot(a, b):
    return jnp.dot(a.astype(jnp.bfloat16), b.astype(jnp.bfloat16),
                   preferred_element_type=jnp.float32)


def _semantic_body(f_ref, w1_ref, b1_ref, w2_ref, b2_ref, lg_ref, p_ref):
    f = f_ref[0]
    h = jnp.maximum(_dot(f, w1_ref[...]) + b1_ref[...], 0.0)
    lg = _dot(h, w2_ref[...]) + b2_ref[...]
    lg_ref[0] = lg
    e = jnp.exp(lg - jnp.max(lg, axis=-1, keepdims=True))
    p_ref[0] = e / jnp.sum(e, axis=-1, keepdims=True)


def _fuse_body(tgt_ref, src_ref, f_ref, p_ref, u1a_ref, u1b_ref, bu1_ref,
               u2_ref, bu2_ref, out_ref):
    t = tgt_ref[0]
    s = src_ref[0]
    t2 = jnp.sum(t * t, axis=1, keepdims=True)
    s2 = jnp.sum(s * s, axis=0, keepdims=True) * 0.25
    cross = _dot(t, s)
    sq = (t2 + s2) + cross

    inf = jnp.float32(jnp.inf)
    zero = jnp.float32(0.0)
    one = jnp.float32(1.0)

    def _weights(v0, v1, v2):
        d0 = jnp.sqrt(jnp.maximum(v0, 1e-12))
        d1 = jnp.sqrt(jnp.maximum(v1, 1e-12))
        d2 = jnp.sqrt(jnp.maximum(v2, 1e-12))
        nd0, nd1, nd2 = -d0, -d1, -d2
        mx = jnp.maximum(nd0, jnp.maximum(nd1, nd2))
        e0 = jnp.exp(nd0 - mx)
        e1 = jnp.exp(nd1 - mx)
        e2 = jnp.exp(nd2 - mx)
        z = e0 + e1 + e2
        return e0 / z, e1 / z, e2 / z

    vmin0 = jnp.min(sq, axis=1, keepdims=True)
    gt0 = sq > vmin0
    vmin1 = jnp.min(jnp.where(gt0, sq, inf), axis=1, keepdims=True)
    gt1 = sq > vmin1
    vmin2 = jnp.min(jnp.where(gt1, sq, inf), axis=1, keepdims=True)
    le2 = sq <= vmin2
    total = jnp.sum(le2.astype(jnp.float32))
    w0, w1, w2 = _weights(vmin0, vmin1, vmin2)
    wsum_fast = jnp.where(gt0, jnp.where(gt1, jnp.where(le2, w2, zero), w1),
                          w0)
    asum_fast = jnp.where(le2, one, zero)

    fused_fast = _dot(wsum_fast, f_ref[0])
    semw_fast = _dot(asum_fast, p_ref[0])

    def _exact_path(_):
        iota = jax.lax.broadcasted_iota(jnp.int32, sq.shape, 1)
        i0 = jnp.min(jnp.where(sq == vmin0, iota, _M), axis=1, keepdims=True)
        m0 = iota == i0
        v1 = jnp.min(jnp.where(m0, inf, sq), axis=1, keepdims=True)
        i1 = jnp.min(jnp.where((sq == v1) & ~m0, iota, _M),
                     axis=1, keepdims=True)
        m01 = m0 | (iota == i1)
        v2 = jnp.min(jnp.where(m01, inf, sq), axis=1, keepdims=True)
        i2 = jnp.min(jnp.where((sq == v2) & ~m01, iota, _M),
                     axis=1, keepdims=True)
        x0, x1, x2 = _weights(vmin0, v1, v2)
        ws = (jnp.where(iota == i0, x0, zero)
              + jnp.where(iota == i1, x1, zero)
              + jnp.where(iota == i2, x2, zero))
        asm = jnp.where(ws > zero, one, zero)
        return _dot(ws, f_ref[0]), _dot(asm, p_ref[0])

    fused, semw3 = jax.lax.cond(
        total == jnp.float32(3 * _TN), lambda _: (fused_fast, semw_fast),
        _exact_path, None)
    semw = semw3 * jnp.float32(1.0 / 3.0)

    h = jnp.maximum(
        _dot(fused, u1a_ref[...]) + _dot(semw, u1b_ref[...])
        + bu1_ref[...], 0.0)
    out_ref[0] = _dot(h, u2_ref[...]) + bu2_ref[...]


def kernel(src_points, tgt_points, src_features, W1, b1, W2, b2,
           U1, bu1, U2, bu2):
    B, M, _ = src_points.shape
    N = tgt_points.shape[1]
    C = src_features.shape[-1]
    NC = W2.shape[-1]
    H2 = U1.shape[-1]
    OUT = U2.shape[-1]

    logits_flat, p_flat = pl.pallas_call(
        _semantic_body,
        grid=(1,),
        in_specs=[
            pl.BlockSpec((1, B * M, C), lambda b: (0, 0, 0)),
            pl.BlockSpec((C, _H1), lambda b: (0, 0)),
            pl.BlockSpec((1, _H1), lambda b: (0, 0)),
            pl.BlockSpec((_H1, NC), lambda b: (0, 0)),
            pl.BlockSpec((1, NC), lambda b: (0, 0)),
        ],
        out_specs=[
            pl.BlockSpec((1, B * M, NC), lambda b: (0, 0, 0)),
            pl.BlockSpec((1, B * M, NC), lambda b: (0, 0, 0)),
        ],
        out_shape=[
            jax.ShapeDtypeStruct((1, B * M, NC), jnp.float32),
            jax.ShapeDtypeStruct((1, B * M, NC), jnp.float32),
        ],
    )(src_features.reshape(1, B * M, C), W1, b1.reshape(1, -1), W2,
      b2.reshape(1, -1))
    logits = logits_flat.reshape(B, M, NC)
    p = p_flat.reshape(B, M, NC)

    tgt_pad = jnp.concatenate(
        [tgt_points, jnp.zeros((B, N, 5), jnp.float32)], axis=2)
    src_pad = jnp.concatenate(
        [src_points.transpose(0, 2, 1) * -2.0,
         jnp.zeros((B, 5, M), jnp.float32)], axis=1)

    upsampled = pl.pallas_call(
        _fuse_body,
        grid=(B, N // _TN),
        in_specs=[
            pl.BlockSpec((1, _TN, 8), lambda b, j: (b, j, 0)),
            pl.BlockSpec((1, 8, M), lambda b, j: (b, 0, 0)),
            pl.BlockSpec((1, M, C), lambda b, j: (b, 0, 0)),
            pl.BlockSpec((1, M, NC), lambda b, j: (b, 0, 0)),
            pl.BlockSpec((C, H2), lambda b, j: (0, 0)),
            pl.BlockSpec((NC, H2), lambda b, j: (0, 0)),
            pl.BlockSpec((1, H2), lambda b, j: (0, 0)),
            pl.BlockSpec((H2, OUT), lambda b, j: (0, 0)),
            pl.BlockSpec((1, OUT), lambda b, j: (0, 0)),
        ],
        out_specs=pl.BlockSpec((1, _TN, OUT), lambda b, j: (b, j, 0)),
        out_shape=jax.ShapeDtypeStruct((B, N, OUT), jnp.float32),
    )(tgt_pad, src_pad, src_features, p, U1[:C], U1[C:],
      bu1.reshape(1, -1), U2, bu2.reshape(1, -1))

    return (upsampled, logits)


# speedup vs baseline: 1.1463x; 1.1077x over previous
import jax
import jax.numpy as jnp
from jax.experimental import pallas as pl

_B, _M, _N, _C = 4, 2048, 4096, 128
_NC, _H1, _H2, _OUT = 20, 128, 256, 128
_TN = 1024

_HIGH = jax.lax.Precision.HIGHEST


def _dot(a, b):
    return jnp.dot(a.astype(jnp.bfloat16), b.astype(jnp.bfloat16),
                   preferred_element_type=jnp.float32)


def _semantic_body(f_ref, w1_ref, b1_ref, w2_ref, b2_ref, lg_ref, p_ref):
    f = f_ref[0]
    h = jnp.maximum(_dot(f, w1_ref[...]) + b1_ref[...], 0.0)
    lg = _dot(h, w2_ref[...]) + b2_ref[...]
    lg_ref[0] = lg
    e = jnp.exp(lg - jnp.max(lg, axis=-1, keepdims=True))
    p_ref[0] = e / jnp.sum(e, axis=-1, keepdims=True)


def _fuse_body(tgt_ref, src_ref, f_ref, p_ref, u1a_ref, u1b_ref, bu1_ref,
               u2_ref, bu2_ref, out_ref):
    t = tgt_ref[0]
    s = src_ref[0]
    t2 = jnp.sum(t * t, axis=1, keepdims=True)
    s2 = jnp.sum(s * s, axis=0, keepdims=True) * 0.25
    cross = _dot(t, s)
    sq = (t2 + s2) + cross

    inf = jnp.float32(jnp.inf)
    zero = jnp.float32(0.0)
    one = jnp.float32(1.0)

    def _weights(v0, v1, v2):
        d0 = jnp.sqrt(jnp.maximum(v0, 1e-12))
        d1 = jnp.sqrt(jnp.maximum(v1, 1e-12))
        d2 = jnp.sqrt(jnp.maximum(v2, 1e-12))
        nd0, nd1, nd2 = -d0, -d1, -d2
        mx = jnp.maximum(nd0, jnp.maximum(nd1, nd2))
        e0 = jnp.exp(nd0 - mx)
        e1 = jnp.exp(nd1 - mx)
        e2 = jnp.exp(nd2 - mx)
        z = e0 + e1 + e2
        return e0 / z, e1 / z, e2 / z

    vmin0 = jnp.min(sq, axis=1, keepdims=True)
    gt0 = sq > vmin0
    vmin1 = jnp.min(jnp.where(gt0, sq, inf), axis=1, keepdims=True)
    gt1 = sq > vmin1
    vmin2 = jnp.min(jnp.where(gt1, sq, inf), axis=1, keepdims=True)
    le2 = sq <= vmin2
    w0, w1, w2 = _weights(vmin0, vmin1, vmin2)
    wsum_fast = jnp.where(gt0, jnp.where(gt1, jnp.where(le2, w2, zero), w1),
                          w0)
    asum_fast = jnp.ceil(wsum_fast)

    fused_fast = _dot(wsum_fast, f_ref[0])
    semw_fast = _dot(asum_fast, p_ref[0])
    total = jnp.sum(semw_fast[:, _NC:])

    def _exact_path(_):
        iota = jax.lax.broadcasted_iota(jnp.int32, sq.shape, 1)
        i0 = jnp.min(jnp.where(sq == vmin0, iota, _M), axis=1, keepdims=True)
        m0 = iota == i0
        v1 = jnp.min(jnp.where(m0, inf, sq), axis=1, keepdims=True)
        i1 = jnp.min(jnp.where((sq == v1) & ~m0, iota, _M),
                     axis=1, keepdims=True)
        m01 = m0 | (iota == i1)
        v2 = jnp.min(jnp.where(m01, inf, sq), axis=1, keepdims=True)
        i2 = jnp.min(jnp.where((sq == v2) & ~m01, iota, _M),
                     axis=1, keepdims=True)
        x0, x1, x2 = _weights(vmin0, v1, v2)
        ws = (jnp.where(iota == i0, x0, zero)
              + jnp.where(iota == i1, x1, zero)
              + jnp.where(iota == i2, x2, zero))
        asm = jnp.where(ws > zero, one, zero)
        return _dot(ws, f_ref[0]), _dot(asm, p_ref[0])

    fused, semw3 = jax.lax.cond(
        total == jnp.float32(3 * _TN), lambda _: (fused_fast, semw_fast),
        _exact_path, None)
    semw = semw3[:, :_NC] * jnp.float32(1.0 / 3.0)

    h = jnp.maximum(
        _dot(fused, u1a_ref[...]) + _dot(semw, u1b_ref[...])
        + bu1_ref[...], 0.0)
    out_ref[0] = _dot(h, u2_ref[...]) + bu2_ref[...]


def kernel(src_points, tgt_points, src_features, W1, b1, W2, b2,
           U1, bu1, U2, bu2):
    B, M, _ = src_points.shape
    N = tgt_points.shape[1]
    C = src_features.shape[-1]
    NC = W2.shape[-1]
    H2 = U1.shape[-1]
    OUT = U2.shape[-1]

    logits_flat, p_flat = pl.pallas_call(
        _semantic_body,
        grid=(1,),
        in_specs=[
            pl.BlockSpec((1, B * M, C), lambda b: (0, 0, 0)),
            pl.BlockSpec((C, _H1), lambda b: (0, 0)),
            pl.BlockSpec((1, _H1), lambda b: (0, 0)),
            pl.BlockSpec((_H1, NC), lambda b: (0, 0)),
            pl.BlockSpec((1, NC), lambda b: (0, 0)),
        ],
        out_specs=[
            pl.BlockSpec((1, B * M, NC), lambda b: (0, 0, 0)),
            pl.BlockSpec((1, B * M, NC), lambda b: (0, 0, 0)),
        ],
        out_shape=[
            jax.ShapeDtypeStruct((1, B * M, NC), jnp.float32),
            jax.ShapeDtypeStruct((1, B * M, NC), jnp.float32),
        ],
    )(src_features.reshape(1, B * M, C), W1, b1.reshape(1, -1), W2,
      b2.reshape(1, -1))
    logits = logits_flat.reshape(B, M, NC)
    p = p_flat.reshape(B, M, NC)

    tgt_pad = jnp.concatenate(
        [tgt_points, jnp.zeros((B, N, 5), jnp.float32)], axis=2)
    src_pad = jnp.concatenate(
        [src_points.transpose(0, 2, 1) * -2.0,
         jnp.zeros((B, 5, M), jnp.float32)], axis=1)

    upsampled = pl.pallas_call(
        _fuse_body,
        grid=(B, N // _TN),
        in_specs=[
            pl.BlockSpec((1, _TN, 8), lambda b, j: (b, j, 0)),
            pl.BlockSpec((1, 8, M), lambda b, j: (b, 0, 0)),
            pl.BlockSpec((1, M, C), lambda b, j: (b, 0, 0)),
            pl.BlockSpec((1, M, NC + 1), lambda b, j: (b, 0, 0)),
            pl.BlockSpec((C, H2), lambda b, j: (0, 0)),
            pl.BlockSpec((NC, H2), lambda b, j: (0, 0)),
            pl.BlockSpec((1, H2), lambda b, j: (0, 0)),
            pl.BlockSpec((H2, OUT), lambda b, j: (0, 0)),
            pl.BlockSpec((1, OUT), lambda b, j: (0, 0)),
        ],
        out_specs=pl.BlockSpec((1, _TN, OUT), lambda b, j: (b, j, 0)),
        out_shape=jax.ShapeDtypeStruct((B, N, OUT), jnp.float32),
    )(tgt_pad, src_pad, src_features,
      jnp.concatenate([p, jnp.ones((B, M, 1), jnp.float32)], axis=2),
      U1[:C], U1[C:], bu1.reshape(1, -1), U2, bu2.reshape(1, -1))

    return (upsampled, logits)
